# baseline (speedup 1.0000x reference)
import functools

import jax
import jax.numpy as jnp
from jax import lax
from jax.experimental import pallas as pl
from jax.experimental.pallas import tpu as pltpu
from jax.experimental.pallas import tpu_sc as plsc

_B = 4096
_D = 32
_V = 1000000

_info = plsc.get_sparse_core_info()
_NC, _NS = _info.num_cores, _info.num_subcores
_NW = _NC * _NS
_BPW = _B // _NW
_G = 16


_EPR = 128 // _D
_VR = _V // _EPR


def _make_gather():
    mesh = plsc.VectorSubcoreMesh(core_axis_name="c", subcore_axis_name="s")
    n_j = _BPW // 16

    @functools.partial(
        pl.kernel,
        mesh=mesh,
        compiler_params=pltpu.CompilerParams(needs_layout_passes=False),
        out_type=(
            jax.ShapeDtypeStruct((_B, _D), jnp.float32),
            jax.ShapeDtypeStruct((_B, _D), jnp.float32),
        ),
        scratch_types=[
            pltpu.VMEM((_BPW,), jnp.int32),
            pltpu.VMEM((_BPW,), jnp.int32),
            pltpu.VMEM((_BPW,), jnp.int32),
            pltpu.VMEM((_BPW,), jnp.int32),
            pltpu.VMEM((_BPW, 128), jnp.float32),
            pltpu.VMEM((_BPW, 128), jnp.float32),
            pltpu.VMEM((_BPW, _D), jnp.float32),
            pltpu.VMEM((_BPW, _D), jnp.float32),
            pltpu.SemaphoreType.DMA,
            pltpu.SemaphoreType.DMA,
        ],
    )
    def gather_k(tgt_hbm, ctx_hbm, in4_hbm, out4_hbm, a_out, b_out,
                 idx_a, idx_b, grp_a, grp_b, rows_a, rows_b, a_v, b_v,
                 sem_a, sem_b):
        wid = lax.axis_index("s") * _NC + lax.axis_index("c")
        base = pl.multiple_of(wid * _BPW, 128)
        pltpu.sync_copy(tgt_hbm.at[pl.ds(base, _BPW)], idx_a)
        pltpu.sync_copy(ctx_hbm.at[pl.ds(base, _BPW)], idx_b)

        def to_grp(k, _):
            off = pl.multiple_of(k * 16, 16)
            grp_a[pl.ds(off, 16)] = lax.shift_right_logical(
                idx_a[pl.ds(off, 16)], 2)
            grp_b[pl.ds(off, 16)] = lax.shift_right_logical(
                idx_b[pl.ds(off, 16)], 2)
            return _

        lax.fori_loop(0, n_j, to_grp, 0)

        ca = pltpu.async_copy(in4_hbm.at[grp_a], rows_a, sem_a)
        cb = pltpu.async_copy(out4_hbm.at[grp_b], rows_b, sem_b)
        ca.wait()
        cb.wait()

        def extract(idx_ref, rows_ref, dst_ref):
            def body(k, _):
                off = pl.multiple_of(k * 16, 16)
                rvec = lax.iota(jnp.int32, 16) + off
                cbase = (idx_ref[pl.ds(off, 16)] & 3) * _D
                for d in range(_D):
                    vals = plsc.load_gather(rows_ref, [rvec, cbase + d])
                    plsc.store_scatter(
                        dst_ref, [rvec, jnp.full((16,), d, jnp.int32)], vals)
                return _

            lax.fori_loop(0, n_j, body, 0)

        extract(idx_a, rows_a, a_v)
        extract(idx_b, rows_b, b_v)
        pltpu.sync_copy(a_v, a_out.at[pl.ds(base, _BPW), :])
        pltpu.sync_copy(b_v, b_out.at[pl.ds(base, _BPW), :])

    return gather_k


_gather = _make_gather()

_BM = 512
_NSTRIPE = 4
_SPS = _B // _NSTRIPE // _BM


def _mm(a_ref, b_ref, o_ref):
    o_ref[...] = lax.dot_general(
        a_ref[...], b_ref[...],
        (((1,), (1,)), ((), ())),
        preferred_element_type=jnp.float32,
    )


def _mm_prev(prev_ref, a_ref, b_ref, o_ref):
    del prev_ref
    _mm(a_ref, b_ref, o_ref)


@functools.cache
def _make_mm(stripe):
    row0 = stripe * _SPS
    ab_specs = [
        pl.BlockSpec((_BM, _D), lambda i: (row0 + i, 0)),
        pl.BlockSpec((_B, _D), lambda i: (0, 0)),
    ]
    out_spec = pl.BlockSpec((_BM, _B), lambda i: (row0 + i, 0))
    out_shape = jax.ShapeDtypeStruct((_B, _B), jnp.float32)
    if stripe == 0:
        return pl.pallas_call(
            _mm,
            grid=(_SPS,),
            in_specs=ab_specs,
            out_specs=out_spec,
            out_shape=out_shape,
        )
    return pl.pallas_call(
        _mm_prev,
        grid=(_SPS,),
        in_specs=[pl.BlockSpec(memory_space=pl.ANY)] + ab_specs,
        out_specs=out_spec,
        out_shape=out_shape,
        input_output_aliases={0: 0},
    )


def kernel(target, context, in_embed, out_embed):
    a, b = _gather(
        target.astype(jnp.int32), context.astype(jnp.int32),
        in_embed.reshape(_VR, 128), out_embed.reshape(_VR, 128),
    )
    scores = _make_mm(0)(a, b)
    for i in range(1, _NSTRIPE):
        scores = _make_mm(i)(scores, a, b)
    return scores


# speedup vs baseline: 8.5472x; 8.5472x over previous
import functools

import jax
import jax.numpy as jnp
from jax import lax
from jax.experimental import pallas as pl
from jax.experimental.pallas import tpu as pltpu
from jax.experimental.pallas import tpu_sc as plsc

_B = 4096
_D = 32
_V = 1000000

_info = plsc.get_sparse_core_info()
_NC, _NS = _info.num_cores, _info.num_subcores
_NW = _NC * _NS
_G = 16
_HALF = _B // 2


def _make_gather(counts):
    n_str = len(counts)
    per = [c // _NW for c in counts]
    mesh = plsc.VectorSubcoreMesh(core_axis_name="c", subcore_axis_name="s")

    @functools.partial(
        pl.kernel,
        mesh=mesh,
        compiler_params=pltpu.CompilerParams(
            use_tc_tiling_on_sc=True, needs_layout_passes=False),
        out_type=tuple(
            jax.ShapeDtypeStruct((c, _D), jnp.float32) for c in counts),
        scratch_types=(
            [pltpu.VMEM((p,), jnp.int32) for p in per]
            + [pltpu.VMEM((p, _D), jnp.float32) for p in per]
            + [
                pltpu.VMEM((_G, _D, 128), jnp.float32),
                pltpu.SemaphoreType.DMA,
                pltpu.SemaphoreType.DMA,
            ]
        ),
    )
    def gather_k(*refs):
        idx_hbm = refs[:n_str]
        tab_hbm = refs[n_str:2 * n_str]
        outs = refs[2 * n_str:3 * n_str]
        idx_v = refs[3 * n_str:4 * n_str]
        col_v = refs[4 * n_str:5 * n_str]
        slab, sem_a, sem_b = refs[5 * n_str:]

        wid = lax.axis_index("s") * _NC + lax.axis_index("c")
        row_lo = lax.iota(jnp.int32, 16)
        row_hi = row_lo + 16

        def phase(idx_ref, src_ref, dst_ref, bpw):
            n_groups = bpw // _G

            def issue(vb, slot, sem):
                l128 = pl.multiple_of((vb >> 7) * 128, 128)
                pltpu.async_copy(
                    src_ref.at[:, pl.ds(l128, 128)], slab.at[slot], sem)

            def extract(vb, j, slot):
                col = jnp.full((16,), vb & 127, jnp.int32)
                jv = jnp.full((16,), j, jnp.int32)
                lo = plsc.load_gather(slab.at[slot], [row_lo, col])
                hi = plsc.load_gather(slab.at[slot], [row_hi, col])
                plsc.store_scatter(dst_ref, [jv, row_lo], lo)
                plsc.store_scatter(dst_ref, [jv, row_hi], hi)

            vv0 = idx_ref[pl.ds(0, _G)]
            for b in range(8):
                issue(vv0[b], b, sem_a)
            for b in range(8, 16):
                issue(vv0[b], b, sem_b)

            def group(g, vcur):
                nxt = jnp.minimum((g + 1) * _G, bpw - _G)
                vnxt = idx_ref[pl.ds(nxt, _G)]
                not_last = g < n_groups - 1
                for half, sem in ((0, sem_a), (1, sem_b)):
                    for b in range(half * 8, half * 8 + 8):
                        pltpu.make_async_copy(
                            src_ref.at[:, pl.ds(0, 128)], slab.at[b], sem).wait()
                    for b in range(half * 8, half * 8 + 8):
                        extract(vcur[b], g * _G + b, b)

                    @pl.when(not_last)
                    def _():
                        for b in range(half * 8, half * 8 + 8):
                            issue(vnxt[b], b, sem)
                return vnxt

            lax.fori_loop(0, n_groups, group, vv0)

        for s in range(n_str):
            base = pl.multiple_of(wid * per[s], _G)
            pltpu.sync_copy(idx_hbm[s].at[pl.ds(base, per[s])], idx_v[s])
            phase(idx_v[s], tab_hbm[s], col_v[s], per[s])
            pltpu.sync_copy(col_v[s], outs[s].at[pl.ds(base, per[s]), :])

    return gather_k


_gather_bc = _make_gather((_B, _HALF))
_gather_a2 = _make_gather((_HALF,))

_BM = 512
_HSTEPS = _HALF // _BM


def _mm(a_ref, b_ref, o_ref):
    o_ref[...] = lax.dot_general(
        a_ref[...], b_ref[...],
        (((1,), (1,)), ((), ())),
        preferred_element_type=jnp.float32,
    )


def _mm_prev(prev_ref, a_ref, b_ref, o_ref):
    del prev_ref
    _mm(a_ref, b_ref, o_ref)


@functools.cache
def _make_mm(half):
    row0 = half * _HSTEPS
    ab_specs = [
        pl.BlockSpec((_BM, _D), lambda i: (i, 0)),
        pl.BlockSpec((_B, _D), lambda i: (0, 0)),
    ]
    out_spec = pl.BlockSpec((_BM, _B), lambda i: (row0 + i, 0))
    out_shape = jax.ShapeDtypeStruct((_B, _B), jnp.float32)
    if half == 0:
        return pl.pallas_call(
            _mm,
            grid=(_HSTEPS,),
            in_specs=ab_specs,
            out_specs=out_spec,
            out_shape=out_shape,
        )
    return pl.pallas_call(
        _mm_prev,
        grid=(_HSTEPS,),
        in_specs=[pl.BlockSpec(memory_space=pl.ANY)] + ab_specs,
        out_specs=out_spec,
        out_shape=out_shape,
        input_output_aliases={0: 0},
    )


def kernel(target, context, in_embed, out_embed):
    tgt = target.astype(jnp.int32)
    ctx = context.astype(jnp.int32)
    inT = in_embed.T
    outT = out_embed.T
    b_g, a1_g = _gather_bc(ctx, tgt[:_HALF], outT, inT)
    (a2_g,) = _gather_a2(tgt[_HALF:], inT)
    scores = _make_mm(0)(a1_g, b_g)
    scores = _make_mm(1)(scores, a2_g, b_g)
    return scores
